# baseline (speedup 1.0000x reference)
import jax
import jax.numpy as jnp
from jax.experimental import pallas as pl

_BLOCK_B = 128


def _add_kernel(x_ref, emb_ref, out_ref):
    out_ref[...] = x_ref[...] + emb_ref[...]


def kernel(x, embedding):
    b, s, d = x.shape
    return pl.pallas_call(
        _add_kernel,
        grid=(b // _BLOCK_B,),
        in_specs=[
            pl.BlockSpec((_BLOCK_B, s, d), lambda i: (i, 0, 0)),
            pl.BlockSpec((1, s, d), lambda i: (0, 0, 0)),
        ],
        out_specs=pl.BlockSpec((_BLOCK_B, s, d), lambda i: (i, 0, 0)),
        out_shape=jax.ShapeDtypeStruct((b, s, d), x.dtype),
    )(x, embedding.reshape(1, s, d))


# speedup vs baseline: 6.0853x; 6.0853x over previous
import jax
import jax.numpy as jnp
from jax.experimental import pallas as pl

_BLK_S = 8


def _add_kernel(x_ref, emb_ref, out_ref):
    out_ref[...] = x_ref[...] + emb_ref[...]


def kernel(x, embedding):
    b, s, d = x.shape
    xt = jnp.transpose(x, (1, 2, 0))
    embt = embedding[:, :, None]
    out = pl.pallas_call(
        _add_kernel,
        grid=(s // _BLK_S,),
        in_specs=[
            pl.BlockSpec((_BLK_S, d, b), lambda i: (i, 0, 0)),
            pl.BlockSpec((_BLK_S, d, 1), lambda i: (i, 0, 0)),
        ],
        out_specs=pl.BlockSpec((_BLK_S, d, b), lambda i: (i, 0, 0)),
        out_shape=jax.ShapeDtypeStruct((s, d, b), x.dtype),
    )(xt, embt)
    return jnp.transpose(out, (2, 0, 1))
